# Initial kernel scaffold; baseline (speedup 1.0000x reference)
import jax
import jax.numpy as jnp
from jax.experimental import pallas as pl


def kernel(x, pos_table):
    raise NotImplementedError("write your pallas kernel here")


# speedup vs baseline: 2.5126x; 2.5126x over previous
import jax
import jax.numpy as jnp
from jax.experimental import pallas as pl


_BLK_S = 512


def _add_kernel(x_ref, pos_ref, o_ref):
    o_ref[...] = x_ref[...] + pos_ref[...]


def kernel(x, pos_table):
    batch, seq_len, dim = x.shape
    blk = _BLK_S
    grid = (seq_len // blk, batch)
    return pl.pallas_call(
        _add_kernel,
        grid=grid,
        in_specs=[
            pl.BlockSpec((1, blk, dim), lambda s, b: (b, s, 0)),
            pl.BlockSpec((blk, dim), lambda s, b: (s, 0)),
        ],
        out_specs=pl.BlockSpec((1, blk, dim), lambda s, b: (b, s, 0)),
        out_shape=jax.ShapeDtypeStruct((batch, seq_len, dim), x.dtype),
    )(x, pos_table)


# speedup vs baseline: 2.6107x; 1.0391x over previous
import jax
import jax.numpy as jnp
from jax.experimental import pallas as pl


_BLK_S = 1024


def _add_kernel(x_ref, pos_ref, o_ref):
    o_ref[...] = x_ref[...] + pos_ref[...]


def kernel(x, pos_table):
    batch, seq_len, dim = x.shape
    blk = _BLK_S
    grid = (seq_len // blk, batch)
    return pl.pallas_call(
        _add_kernel,
        grid=grid,
        in_specs=[
            pl.BlockSpec((1, blk, dim), lambda s, b: (b, s, 0)),
            pl.BlockSpec((blk, dim), lambda s, b: (s, 0)),
        ],
        out_specs=pl.BlockSpec((1, blk, dim), lambda s, b: (b, s, 0)),
        out_shape=jax.ShapeDtypeStruct((batch, seq_len, dim), x.dtype),
    )(x, pos_table)
